# baseline (speedup 1.0000x reference)
import functools

import jax
import jax.numpy as jnp
from jax import lax
from jax.experimental import pallas as pl
from jax.experimental.pallas import tpu as pltpu

_STRIPES = (32, 32, 32, 32)
_LANE = 128


def _tc_body(B, V, stripes, logits_hbm, x_ref, xrep_ref, xcol_ref, out_ref,
             picked, psem, *scratch):
  n = len(stripes)
  bufs = scratch[:n]
  sems = scratch[n]
  offs = [sum(stripes[:k]) for k in range(n)]
  vfull = (V // _LANE) * _LANE
  last_base = vfull - _LANE

  def stripe_copy(k):
    return pltpu.make_async_copy(
        logits_hbm.at[pl.ds(offs[k], stripes[k]), :], bufs[k], sems.at[k])

  def pick_copy(r):
    base = jnp.minimum((x_ref[0, r] // _LANE) * _LANE, last_base)
    base = pl.multiple_of(base, _LANE)
    return pltpu.make_async_copy(
        logits_hbm.at[pl.ds((r // 8) * 8, 8), pl.ds(base, _LANE)],
        picked.at[pl.ds(r * 8, 8), :], psem)

  for k in range(n):
    stripe_copy(k).start()
  for r in range(B):
    pick_copy(r).start()

  ntail = V - vfull
  total = jnp.zeros((1, 1), jnp.float32)
  for k in range(n):
    rk = stripes[k]
    stripe_copy(k).wait()
    chunk = bufs[k][...]
    m = chunk.max(axis=1, keepdims=True)
    s = jnp.exp(chunk - m).sum(axis=1, keepdims=True)
    total = total - jnp.sum(m + jnp.log(s)).reshape(1, 1)
    if ntail:
      tail = chunk[:, V - ntail:]
      tcol = vfull + lax.broadcasted_iota(jnp.int32, (rk, ntail), 1)
      xrows = xcol_ref[offs[k]:offs[k] + rk, :]
      total = total + jnp.sum(
          jnp.where(tcol == xrows, tail, 0.0)).reshape(1, 1)

  for r in range(B):
    pick_copy(r).wait()
  xr = xrep_ref[...]
  baser = jnp.minimum((xr // _LANE) * _LANE, last_base)
  rowi = lax.broadcasted_iota(jnp.int32, (8 * B, _LANE), 0)
  lane = lax.broadcasted_iota(jnp.int32, (8 * B, _LANE), 1)
  rowsel = (rowi % 8) == ((rowi // 8) % 8)
  psel = jnp.where(rowsel & (lane == xr - baser), picked[...], 0.0)
  out_ref[...] = total + jnp.sum(psel).reshape(1, 1)


def kernel(logits, x):
  B, V = logits.shape
  x = x.astype(jnp.int32)
  xrep = jnp.repeat(x, 8).reshape(8 * B, 1)
  out = pl.pallas_call(
      functools.partial(_tc_body, B, V, _STRIPES),
      in_specs=[
          pl.BlockSpec(memory_space=pltpu.MemorySpace.HBM),
          pl.BlockSpec(memory_space=pltpu.MemorySpace.SMEM),
          pl.BlockSpec((8 * B, 1), lambda: (0, 0)),
          pl.BlockSpec((B, 1), lambda: (0, 0)),
      ],
      out_specs=pl.BlockSpec((1, 1), lambda: (0, 0)),
      out_shape=jax.ShapeDtypeStruct((1, 1), jnp.float32),
      scratch_shapes=(
          [pltpu.VMEM((8 * B, _LANE), jnp.float32), pltpu.SemaphoreType.DMA]
          + [pltpu.VMEM((r, V), jnp.float32) for r in _STRIPES]
          + [pltpu.SemaphoreType.DMA((len(_STRIPES),))]
      ),
  )(logits, x.reshape(1, B), xrep, x.reshape(B, 1))
  return out[0, 0]
